# Initial kernel scaffold; baseline (speedup 1.0000x reference)
import functools

import jax
import jax.numpy as jnp
from jax import lax
from jax.experimental import pallas as pl
from jax.experimental.pallas import tpu as pltpu
from jax.experimental.pallas import tpu_sc as plsc

N_VOX = 50000
K_VOL = 27
PAIRS = 25000
C = 64

HALF = N_VOX // 2
TRASH = HALF
ACC_ROWS = HALF + 8
CHUNK = 128
N_SUB = 16
N_PAIRS = K_VOL * PAIRS
N_PAIRS_PAD = 675840
CHUNKS_PER_TILE = N_PAIRS_PAD // CHUNK // N_SUB
INIT_ROWS_PER_TILE = ACC_ROWS // N_SUB
OUT_FULL_CHUNKS = HALF // CHUNK
OUT_TAIL = HALF - OUT_FULL_CHUNKS * CHUNK

MM_BLOCK = 5000


def _mm_body(x_ref, w_ref, y_ref):
    y_ref[...] = jnp.dot(
        x_ref[...], w_ref[0], preferred_element_type=jnp.float32
    )[None]


def _dense_transform(in_feats, weights):
    nb = N_VOX // MM_BLOCK
    return pl.pallas_call(
        _mm_body,
        grid=(nb, K_VOL),
        in_specs=[
            pl.BlockSpec((MM_BLOCK, C), lambda j, k: (j, 0)),
            pl.BlockSpec((1, C, C), lambda j, k: (k, 0, 0)),
        ],
        out_specs=pl.BlockSpec((1, MM_BLOCK, C), lambda j, k: (k, j, 0)),
        out_shape=jax.ShapeDtypeStruct((K_VOL, N_VOX, C), jnp.float32),
    )(in_feats, weights)


def _sc_scatter(y_flat, gidx, omap_flat, bias_row):
    mesh = plsc.VectorSubcoreMesh(core_axis_name="c", subcore_axis_name="s")

    @functools.partial(
        pl.kernel,
        mesh=mesh,
        out_type=jax.ShapeDtypeStruct((N_VOX, C), jnp.float32),
        scratch_types=[
            pltpu.VMEM((CHUNK,), jnp.int32),
            pltpu.VMEM((CHUNK,), jnp.int32),
            pltpu.VMEM((CHUNK,), jnp.int32),
            pltpu.VMEM((CHUNK, C), jnp.float32),
            pltpu.VMEM((CHUNK, C), jnp.float32),
            pltpu.VMEM_SHARED((ACC_ROWS, C), jnp.float32),
            pltpu.SemaphoreType.DMA,
        ],
    )
    def body(y_hbm, gidx_hbm, omap_hbm, bias_hbm, out_hbm,
             gidx_v, omap_v, idx_v, rows_v, bias_v, acc, sem):
        cid = lax.axis_index("c")
        sid = lax.axis_index("s")
        row_base = cid * HALF

        pltpu.sync_copy(bias_hbm, bias_v.at[pl.ds(0, 1)])
        for m in (1, 2, 4, 8, 16, 32, 64):
            pltpu.sync_copy(bias_v.at[pl.ds(0, m)], bias_v.at[pl.ds(m, m)])
        init_base = sid * INIT_ROWS_PER_TILE
        for i in range(INIT_ROWS_PER_TILE // CHUNK):
            pltpu.sync_copy(bias_v, acc.at[pl.ds(init_base + i * CHUNK, CHUNK)])
        rem = INIT_ROWS_PER_TILE % CHUNK
        pltpu.sync_copy(
            bias_v.at[pl.ds(0, rem)],
            acc.at[pl.ds(init_base + INIT_ROWS_PER_TILE - rem, rem)],
        )
        plsc.subcore_barrier()

        def chunk_step(i, _):
            chunk = i * N_SUB + sid
            off = chunk * CHUNK
            pltpu.sync_copy(gidx_hbm.at[pl.ds(off, CHUNK)], gidx_v)
            pltpu.sync_copy(omap_hbm.at[pl.ds(off, CHUNK)], omap_v)
            gather = pltpu.async_copy(y_hbm.at[gidx_v], rows_v, sem)
            for v in range(CHUNK // 16):
                o = omap_v[pl.ds(v * 16, 16)]
                loc = o - row_base
                ok = (loc >= 0) & (loc < HALF)
                idx_v[pl.ds(v * 16, 16)] = jnp.where(ok, loc, TRASH)
            gather.wait()
            pltpu.sync_copy(rows_v, acc.at[idx_v], add=True)
            return 0

        lax.fori_loop(0, CHUNKS_PER_TILE, chunk_step, 0)
        plsc.subcore_barrier()

        for i in range(OUT_FULL_CHUNKS // N_SUB + 1):
            chunk = i * N_SUB + sid
            off = chunk * CHUNK

            @pl.when(chunk < OUT_FULL_CHUNKS)
            def _():
                pltpu.sync_copy(
                    acc.at[pl.ds(off, CHUNK)],
                    out_hbm.at[pl.ds(row_base + off, CHUNK)],
                )

            @pl.when(chunk == OUT_FULL_CHUNKS)
            def _():
                pltpu.sync_copy(
                    acc.at[pl.ds(OUT_FULL_CHUNKS * CHUNK, OUT_TAIL)],
                    out_hbm.at[
                        pl.ds(row_base + OUT_FULL_CHUNKS * CHUNK, OUT_TAIL)
                    ],
                )

    return body(y_flat, gidx, omap_flat, bias_row)


def kernel(in_feats, imap, omap, kernel, bias):
    imap = imap.astype(jnp.int32)
    omap = omap.astype(jnp.int32)

    y = _dense_transform(in_feats, kernel)
    y_flat = y.reshape(K_VOL * N_VOX, C)

    k_off = (jnp.arange(K_VOL, dtype=jnp.int32) * N_VOX)[:, None]
    gidx = (imap + k_off).reshape(-1)
    pad = N_PAIRS_PAD - N_PAIRS
    gidx = jnp.concatenate([gidx, jnp.zeros((pad,), jnp.int32)])
    omap_flat = jnp.concatenate(
        [omap.reshape(-1), jnp.full((pad,), N_VOX, jnp.int32)]
    )
    return _sc_scatter(y_flat, gidx, omap_flat, bias.reshape(1, C))


# speedup vs baseline: 2.9843x; 2.9843x over previous
import functools

import jax
import jax.numpy as jnp
from jax import lax
from jax.experimental import pallas as pl
from jax.experimental.pallas import tpu as pltpu
from jax.experimental.pallas import tpu_sc as plsc

N_VOX = 50000
K_VOL = 27
PAIRS = 25000
C = 64

HALF = N_VOX // 2
TRASH = HALF
ACC_ROWS = HALF + 8
CHUNK = 128
N_SUB = 16
N_PAIRS = K_VOL * PAIRS
N_PAIRS_PAD = 675840
CHUNKS_PER_TILE = N_PAIRS_PAD // CHUNK // N_SUB
INIT_ROWS_PER_TILE = ACC_ROWS // N_SUB
OUT_FULL_CHUNKS = HALF // CHUNK
OUT_TAIL = HALF - OUT_FULL_CHUNKS * CHUNK

MM_BLOCK = 5000


def _mm_body(x_ref, w_ref, y_ref):
    y_ref[...] = jnp.dot(
        x_ref[...], w_ref[0], preferred_element_type=jnp.float32
    )[None]


def _dense_transform(in_feats, weights):
    nb = N_VOX // MM_BLOCK
    return pl.pallas_call(
        _mm_body,
        grid=(nb, K_VOL),
        in_specs=[
            pl.BlockSpec((MM_BLOCK, C), lambda j, k: (j, 0)),
            pl.BlockSpec((1, C, C), lambda j, k: (k, 0, 0)),
        ],
        out_specs=pl.BlockSpec((1, MM_BLOCK, C), lambda j, k: (k, j, 0)),
        out_shape=jax.ShapeDtypeStruct((K_VOL, N_VOX, C), jnp.float32),
    )(in_feats, weights)


def _sc_scatter(y_flat, gidx, omap_flat, bias_row):
    mesh = plsc.VectorSubcoreMesh(core_axis_name="c", subcore_axis_name="s")

    @functools.partial(
        pl.kernel,
        mesh=mesh,
        compiler_params=pltpu.CompilerParams(use_tc_tiling_on_sc=False),
        out_type=jax.ShapeDtypeStruct((N_VOX, C), jnp.float32),
        scratch_types=[
            pltpu.VMEM((CHUNK,), jnp.int32),
            pltpu.VMEM((CHUNK,), jnp.int32),
            pltpu.VMEM((CHUNK,), jnp.int32),
            pltpu.VMEM((CHUNK, C), jnp.float32),
            pltpu.VMEM((CHUNK, C), jnp.float32),
            pltpu.VMEM_SHARED((ACC_ROWS, C), jnp.float32),
            pltpu.SemaphoreType.DMA,
        ],
    )
    def body(y_hbm, gidx_hbm, omap_hbm, bias_hbm, out_hbm,
             gidx_v, omap_v, idx_v, rows_v, bias_v, acc, sem):
        cid = lax.axis_index("c")
        sid = lax.axis_index("s")
        row_base = cid * HALF

        pltpu.sync_copy(bias_hbm, bias_v.at[pl.ds(0, 1)])
        bvals = [bias_v[0, pl.ds(q * 16, 16)] for q in range(C // 16)]
        for r in range(1, CHUNK):
            for q in range(C // 16):
                bias_v[r, pl.ds(q * 16, 16)] = bvals[q]
        init_base = sid * INIT_ROWS_PER_TILE
        for i in range(INIT_ROWS_PER_TILE // CHUNK):
            pltpu.sync_copy(bias_v, acc.at[pl.ds(init_base + i * CHUNK, CHUNK)])
        rem = INIT_ROWS_PER_TILE % CHUNK
        pltpu.sync_copy(
            bias_v.at[pl.ds(0, rem)],
            acc.at[pl.ds(init_base + INIT_ROWS_PER_TILE - rem, rem)],
        )
        plsc.subcore_barrier()

        def chunk_step(i, _):
            chunk = i * N_SUB + sid
            off = chunk * CHUNK
            pltpu.sync_copy(gidx_hbm.at[pl.ds(off, CHUNK)], gidx_v)
            pltpu.sync_copy(omap_hbm.at[pl.ds(off, CHUNK)], omap_v)
            gather = pltpu.async_copy(y_hbm.at[gidx_v], rows_v, sem)
            for v in range(CHUNK // 16):
                o = omap_v[pl.ds(v * 16, 16)]
                loc = o - row_base
                ok = (loc >= 0) & (loc < HALF)
                idx_v[pl.ds(v * 16, 16)] = jnp.where(ok, loc, TRASH)
            gather.wait()
            pltpu.sync_copy(rows_v, acc.at[idx_v], add=True)
            return 0

        lax.fori_loop(0, CHUNKS_PER_TILE, chunk_step, 0)
        plsc.subcore_barrier()

        for i in range(OUT_FULL_CHUNKS // N_SUB + 1):
            chunk = i * N_SUB + sid
            off = chunk * CHUNK

            @pl.when(chunk < OUT_FULL_CHUNKS)
            def _():
                pltpu.sync_copy(
                    acc.at[pl.ds(off, CHUNK)],
                    out_hbm.at[pl.ds(row_base + off, CHUNK)],
                )

            @pl.when(chunk == OUT_FULL_CHUNKS)
            def _():
                pltpu.sync_copy(
                    acc.at[pl.ds(OUT_FULL_CHUNKS * CHUNK, OUT_TAIL)],
                    out_hbm.at[
                        pl.ds(row_base + OUT_FULL_CHUNKS * CHUNK, OUT_TAIL)
                    ],
                )

    return body(y_flat, gidx, omap_flat, bias_row)


def kernel(in_feats, imap, omap, kernel, bias):
    imap = imap.astype(jnp.int32)
    omap = omap.astype(jnp.int32)

    y = _dense_transform(in_feats, kernel)
    y_flat = y.reshape(K_VOL * N_VOX, C)

    k_off = (jnp.arange(K_VOL, dtype=jnp.int32) * N_VOX)[:, None]
    gidx = (imap + k_off).reshape(-1)
    pad = N_PAIRS_PAD - N_PAIRS
    gidx = jnp.concatenate([gidx, jnp.zeros((pad,), jnp.int32)])
    omap_flat = jnp.concatenate(
        [omap.reshape(-1), jnp.full((pad,), N_VOX, jnp.int32)]
    )
    return _sc_scatter(y_flat, gidx, omap_flat, bias.reshape(1, C))


# speedup vs baseline: 6.4096x; 2.1477x over previous
import functools

import jax
import jax.numpy as jnp
from jax import lax
from jax.experimental import pallas as pl
from jax.experimental.pallas import tpu as pltpu
from jax.experimental.pallas import tpu_sc as plsc

N_VOX = 50000
K_VOL = 27
PAIRS = 25000
C = 64

HALF = N_VOX // 2
TRASH = HALF
ACC_ROWS = HALF + 8
CHUNK = 128
N_SUB = 16
N_PAIRS = K_VOL * PAIRS
N_PAIRS_PAD = 675840
CHUNKS_PER_TILE = N_PAIRS_PAD // CHUNK // N_SUB
INIT_ROWS_PER_TILE = ACC_ROWS // N_SUB
OUT_FULL_CHUNKS = HALF // CHUNK
OUT_TAIL = HALF - OUT_FULL_CHUNKS * CHUNK

MM_BLOCK = 5000


def _mm_body(x_ref, w_ref, y_ref):
    w = w_ref[0]
    y_ref[:, :C] = jnp.dot(
        x_ref[:, :C], w, preferred_element_type=jnp.float32
    )
    y_ref[:, C:] = jnp.dot(
        x_ref[:, C:], w, preferred_element_type=jnp.float32
    )


def _dense_transform(in2, weights):
    nb = (N_VOX // 2) // MM_BLOCK
    return pl.pallas_call(
        _mm_body,
        grid=(nb, K_VOL),
        in_specs=[
            pl.BlockSpec((MM_BLOCK, 2 * C), lambda j, k: (j, 0)),
            pl.BlockSpec((1, C, C), lambda j, k: (k, 0, 0)),
        ],
        out_specs=pl.BlockSpec((MM_BLOCK, 2 * C), lambda j, k: (k * nb + j, 0)),
        out_shape=jax.ShapeDtypeStruct((K_VOL * N_VOX // 2, 2 * C), jnp.float32),
    )(in2, weights)


def _sc_scatter(y_flat, gidx, omap_flat, bias_row):
    mesh = plsc.VectorSubcoreMesh(core_axis_name="c", subcore_axis_name="s")

    @functools.partial(
        pl.kernel,
        mesh=mesh,
        compiler_params=pltpu.CompilerParams(use_tc_tiling_on_sc=False),
        out_type=jax.ShapeDtypeStruct((N_VOX, C), jnp.float32),
        scratch_types=[
            pltpu.VMEM((CHUNK,), jnp.int32),
            pltpu.VMEM((CHUNK,), jnp.int32),
            pltpu.VMEM((CHUNK,), jnp.int32),
            pltpu.VMEM((CHUNK,), jnp.int32),
            pltpu.VMEM((CHUNK,), jnp.int32),
            pltpu.VMEM((CHUNK, C), jnp.float32),
            pltpu.VMEM((CHUNK, C), jnp.float32),
            pltpu.VMEM((CHUNK, C), jnp.float32),
            pltpu.VMEM_SHARED((ACC_ROWS, C), jnp.float32),
            pltpu.SemaphoreType.DMA,
            pltpu.SemaphoreType.DMA,
        ],
    )
    def body(y_hbm, gidx_hbm, omap_hbm, bias_hbm, out_hbm,
             gidx0, gidx1, omap0, omap1, idx_v, rows0, rows1, bias_v, acc,
             sem0, sem1):
        cid = lax.axis_index("c")
        sid = lax.axis_index("s")
        row_base = cid * HALF
        gidx_b = (gidx0, gidx1)
        omap_b = (omap0, omap1)
        rows_b = (rows0, rows1)
        sem_b = (sem0, sem1)

        pltpu.sync_copy(bias_hbm, bias_v.at[pl.ds(0, 1)])
        bvals = [bias_v[0, pl.ds(q * 16, 16)] for q in range(C // 16)]
        for r in range(1, CHUNK):
            for q in range(C // 16):
                bias_v[r, pl.ds(q * 16, 16)] = bvals[q]
        init_base = sid * INIT_ROWS_PER_TILE
        for i in range(INIT_ROWS_PER_TILE // CHUNK):
            pltpu.sync_copy(bias_v, acc.at[pl.ds(init_base + i * CHUNK, CHUNK)])
        rem = INIT_ROWS_PER_TILE % CHUNK
        pltpu.sync_copy(
            bias_v.at[pl.ds(0, rem)],
            acc.at[pl.ds(init_base + INIT_ROWS_PER_TILE - rem, rem)],
        )
        plsc.subcore_barrier()

        def load_and_fire(i, b):
            off = (i * N_SUB + sid) * CHUNK
            pltpu.sync_copy(gidx_hbm.at[pl.ds(off, CHUNK)], gidx_b[b])
            pltpu.sync_copy(omap_hbm.at[pl.ds(off, CHUNK)], omap_b[b])
            pltpu.make_async_copy(
                y_hbm.at[gidx_b[b]], rows_b[b], sem_b[b]
            ).start()

        def drain_and_scatter(b):
            pltpu.make_async_copy(
                y_hbm.at[gidx_b[b]], rows_b[b], sem_b[b]
            ).wait()
            for v in range(CHUNK // 16):
                o = omap_b[b][pl.ds(v * 16, 16)]
                loc = o - row_base
                ok = (loc >= 0) & (loc < HALF)
                idx_v[pl.ds(v * 16, 16)] = jnp.where(ok, loc, TRASH)
            pltpu.sync_copy(rows_b[b], acc.at[idx_v], add=True)

        load_and_fire(0, 0)

        def pair_step(i2, _):
            for b in (0, 1):
                i = i2 * 2 + b

                @pl.when(i + 1 < CHUNKS_PER_TILE)
                def _():
                    load_and_fire(i + 1, 1 - b)

                drain_and_scatter(b)
            return 0

        lax.fori_loop(0, CHUNKS_PER_TILE // 2, pair_step, 0)
        plsc.subcore_barrier()

        for i in range(OUT_FULL_CHUNKS // N_SUB + 1):
            chunk = i * N_SUB + sid
            off = chunk * CHUNK

            @pl.when(chunk < OUT_FULL_CHUNKS)
            def _():
                pltpu.sync_copy(
                    acc.at[pl.ds(off, CHUNK)],
                    out_hbm.at[pl.ds(row_base + off, CHUNK)],
                )

            @pl.when(chunk == OUT_FULL_CHUNKS)
            def _():
                pltpu.sync_copy(
                    acc.at[pl.ds(OUT_FULL_CHUNKS * CHUNK, OUT_TAIL)],
                    out_hbm.at[
                        pl.ds(row_base + OUT_FULL_CHUNKS * CHUNK, OUT_TAIL)
                    ],
                )

    return body(y_flat, gidx, omap_flat, bias_row)


def kernel(in_feats, imap, omap, kernel, bias):
    imap = imap.astype(jnp.int32)
    omap = omap.astype(jnp.int32)

    in2 = in_feats.reshape(N_VOX // 2, 2 * C)
    y128 = _dense_transform(in2, kernel)
    y_flat = y128.reshape(K_VOL * N_VOX, C)

    k_off = (jnp.arange(K_VOL, dtype=jnp.int32) * N_VOX)[:, None]
    gidx = (imap + k_off).reshape(-1)
    pad = N_PAIRS_PAD - N_PAIRS
    gidx = jnp.concatenate([gidx, jnp.zeros((pad,), jnp.int32)])
    omap_flat = jnp.concatenate(
        [omap.reshape(-1), jnp.full((pad,), N_VOX, jnp.int32)]
    )
    return _sc_scatter(y_flat, gidx, omap_flat, bias.reshape(1, C))
